# Initial kernel scaffold; baseline (speedup 1.0000x reference)
import jax
import jax.numpy as jnp
from jax.experimental import pallas as pl


def kernel(pred_logits, pred_boxes, orig_target_sizes):
    raise NotImplementedError("write your pallas kernel here")


# speedup vs baseline: 11.4707x; 11.4707x over previous
import functools

import jax
import jax.numpy as jnp
from jax.experimental import pallas as pl

NUM_CLASSES = 80
K = 300
G = 128
NG = 12500
Q = 512


def _groupmax_body(x_ref, o_ref):
    x = x_ref[0]
    o_ref[0, 0, :] = jnp.max(x, axis=1)


def _groupmax(flat_logits):
    B = flat_logits.shape[0]
    x3 = flat_logits.reshape(B, NG, G)
    out = pl.pallas_call(
        _groupmax_body,
        grid=(B,),
        in_specs=[pl.BlockSpec((1, NG, G), lambda b: (b, 0, 0))],
        out_specs=pl.BlockSpec((1, 1, NG), lambda b: (b, 0, 0)),
        out_shape=jax.ShapeDtypeStruct((B, 1, NG), jnp.float32),
    )(x3)
    return out.reshape(B, NG)


def kernel(pred_logits, pred_boxes, orig_target_sizes):
    B, N, C = pred_logits.shape
    flat = pred_logits.reshape(B, N * C)

    gmax = _groupmax(flat)

    _, top_g = jax.lax.top_k(gmax, Q)
    top_g = jnp.sort(top_g, axis=1)

    cand = jnp.take_along_axis(
        pred_logits.reshape(B, NG, G),
        jnp.broadcast_to(top_g[:, :, None], (B, Q, G)), axis=1)
    scores = jax.nn.sigmoid(cand).reshape(B, Q * G)
    flat_ids = (top_g[:, :, None] * G
                + jnp.arange(G, dtype=jnp.int32)[None, None, :]).reshape(B, Q * G)

    top_scores, pos = jax.lax.top_k(scores, K)
    flat_idx = jnp.take_along_axis(flat_ids, pos, axis=1)
    labels = flat_idx % NUM_CLASSES
    qidx = flat_idx // NUM_CLASSES

    scale = jnp.tile(orig_target_sizes, (1, 2))[:, None, :]
    cx, cy, w, h = jnp.split(pred_boxes, 4, axis=-1)
    xyxy = jnp.concatenate(
        [cx - 0.5 * w, cy - 0.5 * h, cx + 0.5 * w, cy + 0.5 * h], axis=-1)
    bbox = xyxy * scale
    final_boxes = jnp.take_along_axis(
        bbox, jnp.broadcast_to(qidx[:, :, None], (B, K, 4)), axis=1)
    return (labels, final_boxes, top_scores)
